# baseline (speedup 1.0000x reference)
import functools

import jax
import jax.numpy as jnp
from jax import lax
from jax.experimental import pallas as pl
from jax.experimental.pallas import tpu as pltpu, tpu_sc as plsc

V = 1000
H = 32
DP = 128
LP = 24

_BB = 128
_NCHUNK = 2


@functools.cache
def _make_sc_gather(n_rows: int):
    info = plsc.get_sparse_core_info()
    nc, ns = info.num_cores, info.num_subcores
    nw = nc * ns
    assert n_rows % nw == 0
    b_per_w = n_rows // nw
    mesh = plsc.VectorSubcoreMesh(core_axis_name="c", subcore_axis_name="s")

    @functools.partial(
        pl.kernel, mesh=mesh,
        compiler_params=pltpu.CompilerParams(use_tc_tiling_on_sc=True),
        out_type=jax.ShapeDtypeStruct((n_rows, DP), jnp.float32),
        scratch_types=[
            pltpu.VMEM((b_per_w,), jnp.int32),
            pltpu.VMEM((b_per_w, DP), jnp.float32),
            pltpu.SemaphoreType.DMA,
        ],
    )
    def gather_k(idx_hbm, table_hbm, out_hbm, idx_v, rows_v, sem):
        wid = lax.axis_index("s") * nc + lax.axis_index("c")
        base = wid * b_per_w
        pltpu.sync_copy(idx_hbm.at[pl.ds(base, b_per_w)], idx_v)
        pltpu.async_copy(table_hbm.at[idx_v], rows_v, sem).wait()
        pltpu.sync_copy(rows_v, out_hbm.at[pl.ds(base, b_per_w)])

    return gather_k


def _prep_kernel(inp_ref, emb_ref, w1_ref, ipad_ref, emb_p_ref, w1_p_ref):
    b = inp_ref.shape[0]
    filler = jax.lax.broadcasted_iota(jnp.int32, (b, LP), 0) % V
    ipad_ref[...] = jnp.concatenate(
        [inp_ref[...], filler[:, : LP - inp_ref.shape[1]]], axis=1)
    emb_p_ref[...] = jnp.zeros_like(emb_p_ref)
    emb_p_ref[:, :H] = emb_ref[...]
    w1_p_ref[...] = jnp.zeros_like(w1_p_ref)
    w1_p_ref[:H, :] = w1_ref[...]


def _prep(inputs_i32, emb, W1):
    b = inputs_i32.shape[0]
    return pl.pallas_call(
        _prep_kernel,
        out_shape=(
            jax.ShapeDtypeStruct((b, LP), jnp.int32),
            jax.ShapeDtypeStruct((V, DP), jnp.float32),
            jax.ShapeDtypeStruct((DP, H), jnp.float32),
        ),
    )(inputs_i32, emb, W1)


def _mlp_kernel(x_ref, w1_ref, b1_ref, w2_ref, b2_ref, out_ref):
    h = jnp.maximum(
        jnp.dot(x_ref[...], w1_ref[...], preferred_element_type=jnp.float32)
        + b1_ref[...],
        0.0)
    y = jnp.dot(h, w2_ref[...], preferred_element_type=jnp.float32) + b2_ref[...]
    bb, l, v = out_ref.shape
    out_ref[...] = y.reshape(bb, LP, v)[:, :l, :]


def _mlp_chunk_kernel(prev_ref, x_ref, w1_ref, b1_ref, w2_ref, b2_ref,
                      out_ref):
    del prev_ref
    _mlp_kernel(x_ref, w1_ref, b1_ref, w2_ref, b2_ref, out_ref)


def kernel(inputs, emb, W1, b1, W2, b2):
    B, L = inputs.shape
    bc = B // _NCHUNK

    ipad, emb_p, w1_p = _prep(inputs.astype(jnp.int32), emb, W1)
    idx_flat = ipad.reshape(B * LP)

    gather = _make_sc_gather(bc * LP)
    xs = [gather(idx_flat[c * bc * LP:(c + 1) * bc * LP], emb_p)
          for c in range(_NCHUNK)]

    b1r, b2r = b1.reshape(1, H), b2.reshape(1, V)
    nblk = bc // _BB
    out = None
    for c in range(_NCHUNK):
        blk0 = c * nblk
        if out is None:
            operands, in_specs, aliases = [], [], {}
        else:
            operands = [out]
            in_specs = [pl.BlockSpec(memory_space=pl.ANY)]
            aliases = {0: 0}
        operands += [xs[c], w1_p, b1r, W2, b2r]
        in_specs += [
            pl.BlockSpec((_BB * LP, DP), lambda g: (g, 0)),
            pl.BlockSpec((DP, H), lambda g: (0, 0)),
            pl.BlockSpec((1, H), lambda g: (0, 0)),
            pl.BlockSpec((H, V), lambda g: (0, 0)),
            pl.BlockSpec((1, V), lambda g: (0, 0)),
        ]
        out = pl.pallas_call(
            _mlp_kernel if out is None else _mlp_chunk_kernel,
            grid=(nblk,),
            in_specs=in_specs,
            out_specs=pl.BlockSpec((_BB, L, V),
                                   lambda g, b0=blk0: (b0 + g, 0, 0)),
            out_shape=jax.ShapeDtypeStruct((B, L, V), jnp.float32),
            input_output_aliases=aliases,
        )(*operands)
    return out


# speedup vs baseline: 1.0234x; 1.0234x over previous
import functools

import jax
import jax.numpy as jnp
from jax import lax
from jax.experimental import pallas as pl
from jax.experimental.pallas import tpu as pltpu, tpu_sc as plsc

V = 1000
H = 32
DP = 128
LP = 24

_BB = 128


@functools.cache
def _make_sc_gather(n_rows: int):
    info = plsc.get_sparse_core_info()
    nc, ns = info.num_cores, info.num_subcores
    nw = nc * ns
    assert n_rows % nw == 0
    b_per_w = n_rows // nw
    mesh = plsc.VectorSubcoreMesh(core_axis_name="c", subcore_axis_name="s")

    @functools.partial(
        pl.kernel, mesh=mesh,
        compiler_params=pltpu.CompilerParams(use_tc_tiling_on_sc=True),
        out_type=jax.ShapeDtypeStruct((n_rows, DP), jnp.float32),
        scratch_types=[
            pltpu.VMEM((b_per_w,), jnp.int32),
            pltpu.VMEM((b_per_w, DP), jnp.float32),
            pltpu.SemaphoreType.DMA,
        ],
    )
    def gather_k(idx_hbm, table_hbm, out_hbm, idx_v, rows_v, sem):
        wid = lax.axis_index("s") * nc + lax.axis_index("c")
        base = wid * b_per_w
        pltpu.sync_copy(idx_hbm.at[pl.ds(base, b_per_w)], idx_v)
        pltpu.async_copy(table_hbm.at[idx_v], rows_v, sem).wait()
        pltpu.sync_copy(rows_v, out_hbm.at[pl.ds(base, b_per_w)])

    return gather_k


def _prep_kernel(inp_ref, emb_ref, w1_ref, ipad_ref, emb_p_ref, w1_p_ref):
    b = inp_ref.shape[0]
    filler = jax.lax.broadcasted_iota(jnp.int32, (b, LP), 0) % V
    ipad_ref[...] = jnp.concatenate(
        [inp_ref[...], filler[:, : LP - inp_ref.shape[1]]], axis=1)
    emb_p_ref[...] = jnp.zeros_like(emb_p_ref)
    emb_p_ref[:, :H] = emb_ref[...]
    w1_p_ref[...] = jnp.zeros_like(w1_p_ref)
    w1_p_ref[:H, :] = w1_ref[...]


def _prep(inputs_i32, emb, W1):
    b = inputs_i32.shape[0]
    return pl.pallas_call(
        _prep_kernel,
        out_shape=(
            jax.ShapeDtypeStruct((b, LP), jnp.int32),
            jax.ShapeDtypeStruct((V, DP), jnp.float32),
            jax.ShapeDtypeStruct((DP, H), jnp.float32),
        ),
    )(inputs_i32, emb, W1)


def _mlp_kernel(x_ref, w1_ref, b1_ref, w2_ref, b2_ref, out_ref):
    h = jnp.maximum(
        jnp.dot(x_ref[...], w1_ref[...], preferred_element_type=jnp.float32)
        + b1_ref[...],
        0.0)
    y = jnp.dot(h, w2_ref[...], preferred_element_type=jnp.float32) + b2_ref[...]
    bb, l, v = out_ref.shape
    out_ref[...] = y.reshape(bb, LP, v)[:, :l, :]


def kernel(inputs, emb, W1, b1, W2, b2):
    B, L = inputs.shape

    ipad, emb_p, w1_p = _prep(inputs.astype(jnp.int32), emb, W1)

    x = _make_sc_gather(B * LP)(ipad.reshape(B * LP), emb_p)

    out = pl.pallas_call(
        _mlp_kernel,
        grid=(B // _BB,),
        in_specs=[
            pl.BlockSpec((_BB * LP, DP), lambda g: (g, 0)),
            pl.BlockSpec((DP, H), lambda g: (0, 0)),
            pl.BlockSpec((1, H), lambda g: (0, 0)),
            pl.BlockSpec((H, V), lambda g: (0, 0)),
            pl.BlockSpec((1, V), lambda g: (0, 0)),
        ],
        out_specs=pl.BlockSpec((_BB, L, V), lambda g: (g, 0, 0)),
        out_shape=jax.ShapeDtypeStruct((B, L, V), jnp.float32),
    )(x, w1_p, b1.reshape(1, H), W2, b2.reshape(1, V))
    return out
